# Initial kernel scaffold; baseline (speedup 1.0000x reference)
import jax
import jax.numpy as jnp
from jax.experimental import pallas as pl


def kernel(x, edge_index, ggnn_w, gru_w_ih, gru_w_hh, gru_b_ih, gru_b_hh, conv1_w, conv1_b, conv2_w, conv2_b, cconv1_w, cconv1_b, cconv2_w, cconv2_b, bn1_g, bn1_b, bnc_g, bnc_b, mlp_y_w, mlp_y_b, mlp_z_w, mlp_z_b):
    raise NotImplementedError("write your pallas kernel here")


# speedup vs baseline: 4.3568x; 4.3568x over previous
import functools

import jax
import jax.numpy as jnp
from jax import lax
from jax.experimental import pallas as pl
from jax.experimental.pallas import tpu as pltpu
from jax.experimental.pallas import tpu_sc as plsc

_N = 10000
_E = 320000
_IN = 128
_D = 256
_HD = 128
_NC = 2
_NS = 16
_EB = 100
_EPT = _E // _NS
_NB = _EPT // _EB
_NP = 10240
_RPT = _NP // _NS
_STEPS = 6


_CB = 40
_NCHUNK = _NB // _CB


def _sc_scatter_body(m_hbm, gidx_hbm, dst_hbm, zeros_hbm, out_hbm,
                     gidx_v, dst_v, rows_v, agg_sh, sem):
    c = lax.axis_index("c")
    s = lax.axis_index("s")
    pltpu.sync_copy(zeros_hbm, agg_sh.at[pl.ds(s * _RPT, _RPT)])
    plsc.subcore_barrier()

    def chunk_body(k, carry):
        pltpu.sync_copy(gidx_hbm.at[c, pl.ds(s * _NB + k * _CB, _CB)], gidx_v)
        pltpu.sync_copy(dst_hbm.at[pl.ds(s * _NB + k * _CB, _CB)], dst_v)

        def body(j, carry2):
            pltpu.async_copy(m_hbm.at[gidx_v.at[j]], rows_v, sem).wait()
            pltpu.sync_copy(rows_v, agg_sh.at[dst_v.at[j]], add=True)
            return carry2

        return lax.fori_loop(0, _CB, body, carry)

    lax.fori_loop(0, _NCHUNK, chunk_body, 0)
    plsc.subcore_barrier()
    pltpu.sync_copy(agg_sh.at[pl.ds(s * _RPT, _RPT)],
                    out_hbm.at[c, pl.ds(s * _RPT, _RPT)])


_sc_scatter = pl.kernel(
    _sc_scatter_body,
    out_type=jax.ShapeDtypeStruct((_NC, _NP, _HD), jnp.float32),
    mesh=plsc.VectorSubcoreMesh(core_axis_name="c", subcore_axis_name="s"),
    scratch_types=[
        pltpu.VMEM((_CB, _EB), jnp.int32),
        pltpu.VMEM((_CB, _EB), jnp.int32),
        pltpu.VMEM((_EB, _HD), jnp.float32),
        pltpu.VMEM_SHARED((_NP, _HD), jnp.float32),
        pltpu.SemaphoreType.DMA,
    ],
)


def kernel(x, edge_index, ggnn_w, gru_w_ih, gru_w_hh, gru_b_ih, gru_b_hh,
           conv1_w, conv1_b, conv2_w, conv2_b,
           cconv1_w, cconv1_b, cconv2_w, cconv2_b,
           bn1_g, bn1_b, bnc_g, bnc_b,
           mlp_y_w, mlp_y_b, mlp_z_w, mlp_z_b):
    src = edge_index[0]
    dst = edge_index[1]
    gidx = (src[None, :] * 2 + jnp.arange(2, dtype=jnp.int32)[:, None])
    gidx = gidx.reshape(_NC, _NS * _NB, _EB)
    dstr = dst.reshape(_NS * _NB, _EB)
    zeros = jnp.zeros((_RPT, _HD), jnp.float32)

    h = jnp.pad(x, ((0, 0), (0, _D - _IN)))
    for i in range(_STEPS):
        m = h @ ggnn_w[i]
        agg2 = _sc_scatter(m.reshape(2 * _N, _HD), gidx, dstr, zeros)
        agg = jnp.concatenate([agg2[0, :_N], agg2[1, :_N]], axis=1)
        gi = agg @ gru_w_ih.T + gru_b_ih
        gh = h @ gru_w_hh.T + gru_b_hh
        i_r, i_z, i_n = jnp.split(gi, 3, axis=1)
        h_r, h_z, h_n = jnp.split(gh, 3, axis=1)
        r = jax.nn.sigmoid(i_r + h_r)
        z = jax.nn.sigmoid(i_z + h_z)
        nn_ = jnp.tanh(i_n + r * h_n)
        h = (1.0 - z) * nn_ + z * h

    outputs = h
    x_i = x[None]
    h_i = outputs[None]
    c_i = jnp.concatenate((h_i, x_i), axis=-1)

    def _conv1d(t, w, b):
        out = lax.conv_general_dilated(t, w, window_strides=(1,), padding='VALID',
                                       dimension_numbers=('NCH', 'OIH', 'NCH'))
        return out + b[None, :, None]

    def _batchnorm1d(t, g, b, eps=1e-5):
        mean = jnp.mean(t, axis=(0, 2), keepdims=True)
        var = jnp.var(t, axis=(0, 2), keepdims=True)
        return (t - mean) / jnp.sqrt(var + eps) * g[None, :, None] + b[None, :, None]

    def _maxpool1d(t, k, st):
        return lax.reduce_window(t, -jnp.inf, lax.max, (1, 1, k), (1, 1, st), 'VALID')

    h_t = jnp.transpose(h_i, (0, 2, 1))
    Y1 = _maxpool1d(jax.nn.relu(_batchnorm1d(_conv1d(h_t, conv1_w, conv1_b), bn1_g, bn1_b)), 3, 2)
    Y2 = jnp.transpose(_maxpool1d(jax.nn.relu(_batchnorm1d(_conv1d(Y1, conv2_w, conv2_b), bn1_g, bn1_b)), 2, 2), (0, 2, 1))

    c_t = jnp.transpose(c_i, (0, 2, 1))
    Z1 = _maxpool1d(jax.nn.relu(_batchnorm1d(_conv1d(c_t, cconv1_w, cconv1_b), bnc_g, bnc_b)), 3, 2)
    Z2 = jnp.transpose(_maxpool1d(jax.nn.relu(_batchnorm1d(_conv1d(Z1, cconv2_w, cconv2_b), bnc_g, bnc_b)), 2, 2), (0, 2, 1))

    before_avg = (Y2 @ mlp_y_w.T + mlp_y_b) * (Z2 @ mlp_z_w.T + mlp_z_b)
    avg = before_avg.mean(axis=1)
    return jax.nn.sigmoid(avg)
